# baseline (speedup 1.0000x reference)
import functools

import jax
import jax.numpy as jnp
from jax import lax
from jax.experimental import pallas as pl
from jax.experimental.pallas import tpu as pltpu
from jax.experimental.pallas import tpu_sc as plsc

K = 16
CHUNK = 128


@functools.partial(jax.jit, static_argnames=("num_cores", "num_subcores"))
def _mf_sc(user_idx, prob_idx, user_table_t, prob_table_t, w16, b16,
           num_cores, num_subcores):
    batch = user_idx.shape[0]
    nw = num_cores * num_subcores
    bpw = batch // nw
    n_chunks = bpw // CHUNK

    idx_u2 = user_idx.reshape(nw * n_chunks, CHUNK)
    idx_p2 = prob_idx.reshape(nw * n_chunks, CHUNK)

    mesh = plsc.VectorSubcoreMesh(
        core_axis_name="c", subcore_axis_name="s",
        num_cores=num_cores, num_subcores=num_subcores)

    @functools.partial(
        pl.kernel,
        out_type=jax.ShapeDtypeStruct((batch,), jnp.float32),
        mesh=mesh,
        scratch_types=[
            pltpu.VMEM((n_chunks, CHUNK), jnp.int32),
            pltpu.VMEM((n_chunks, CHUNK), jnp.int32),
            pltpu.VMEM((K, bpw), jnp.float32),
            pltpu.VMEM((K, bpw), jnp.float32),
            pltpu.VMEM((K,), jnp.float32),
            pltpu.VMEM((K,), jnp.float32),
            pltpu.VMEM((bpw,), jnp.float32),
            pltpu.SemaphoreType.DMA,
        ],
        compiler_params=pltpu.CompilerParams(
            needs_layout_passes=False, use_tc_tiling_on_sc=False),
    )
    def mf_kernel(idx_u_hbm, idx_p_hbm, ut_hbm, pt_hbm, w_hbm, b_hbm,
                  out_hbm, idx_u_v, idx_p_v, vals_u, vals_p, w_v, b_v,
                  out_v, sem):
        wid = lax.axis_index("s") * num_cores + lax.axis_index("c")
        base = wid * bpw

        pltpu.sync_copy(idx_u_hbm.at[pl.ds(wid * n_chunks, n_chunks)], idx_u_v)
        pltpu.sync_copy(idx_p_hbm.at[pl.ds(wid * n_chunks, n_chunks)], idx_p_v)
        pltpu.sync_copy(w_hbm, w_v)
        pltpu.sync_copy(b_hbm, b_v)

        copies = []
        for k in range(K):
            for c in range(n_chunks):
                copies.append(pltpu.async_copy(
                    ut_hbm.at[k].at[idx_u_v.at[c]],
                    vals_u.at[k, pl.ds(c * CHUNK, CHUNK)], sem))
                copies.append(pltpu.async_copy(
                    pt_hbm.at[k].at[idx_p_v.at[c]],
                    vals_p.at[k, pl.ds(c * CHUNK, CHUNK)], sem))
        for c in copies:
            c.wait()

        wv = w_v[...]
        bv = b_v[...]

        def body(g, carry):
            col = g * K
            acc = bv
            for k in range(K):
                acc = acc + (vals_u[k, pl.ds(col, K)]
                             * vals_p[k, pl.ds(col, K)]) * wv[k]
            out_v[pl.ds(col, K)] = acc
            return carry

        lax.fori_loop(0, bpw // K, body, 0)

        pltpu.sync_copy(out_v, out_hbm.at[pl.ds(base, bpw)])

    return mf_kernel(idx_u2, idx_p2, user_table_t, prob_table_t, w16, b16)


def kernel(input_user, input_prob, user_table, prob_table, dense_w, dense_b):
    info = plsc.get_sparse_core_info()
    out = _mf_sc(
        input_user.reshape(-1),
        input_prob.reshape(-1),
        user_table.T,
        prob_table.T,
        dense_w.reshape(K),
        jnp.broadcast_to(dense_b, (K,)),
        info.num_cores,
        info.num_subcores,
    )
    return out.reshape(-1, 1)


# speedup vs baseline: 2.6698x; 2.6698x over previous
import functools

import jax
import jax.numpy as jnp
from jax import lax
from jax.experimental import pallas as pl
from jax.experimental.pallas import tpu as pltpu
from jax.experimental.pallas import tpu_sc as plsc

K = 16
CHUNK = 128


@functools.partial(jax.jit, static_argnames=("num_cores", "num_subcores"))
def _mf_sc(user_idx, prob_idx, user_sup, prob_sup, w16, b16,
           num_cores, num_subcores):
    batch = user_idx.shape[0]
    nw = num_cores * num_subcores
    bpw = batch // nw
    n_chunks = bpw // CHUNK

    idx_u2 = user_idx.reshape(nw * n_chunks, CHUNK)
    idx_p2 = prob_idx.reshape(nw * n_chunks, CHUNK)

    mesh = plsc.VectorSubcoreMesh(
        core_axis_name="c", subcore_axis_name="s",
        num_cores=num_cores, num_subcores=num_subcores)

    @functools.partial(
        pl.kernel,
        out_type=jax.ShapeDtypeStruct((batch,), jnp.float32),
        mesh=mesh,
        scratch_types=[
            pltpu.VMEM((n_chunks, CHUNK), jnp.int32),
            pltpu.VMEM((n_chunks, CHUNK), jnp.int32),
            pltpu.VMEM((n_chunks, CHUNK), jnp.int32),
            pltpu.VMEM((n_chunks, CHUNK), jnp.int32),
            pltpu.VMEM((CHUNK, 128), jnp.float32),
            pltpu.VMEM((CHUNK, 128), jnp.float32),
            pltpu.VMEM((CHUNK, 128), jnp.float32),
            pltpu.VMEM((CHUNK, 128), jnp.float32),
            pltpu.VMEM((K,), jnp.float32),
            pltpu.VMEM((K,), jnp.float32),
            pltpu.VMEM((bpw,), jnp.float32),
            pltpu.SemaphoreType.DMA,
            pltpu.SemaphoreType.DMA,
        ],
        compiler_params=pltpu.CompilerParams(
            needs_layout_passes=False, use_tc_tiling_on_sc=False),
    )
    def mf_kernel(idx_u_hbm, idx_p_hbm, ut_hbm, pt_hbm, w_hbm, b_hbm,
                  out_hbm, idx_u_v, idx_p_v, sup_u_v, sup_p_v,
                  bu0, bu1, bp0, bp1, w_v, b_v, out_v, sem0, sem1):
        wid = lax.axis_index("s") * num_cores + lax.axis_index("c")
        base = wid * bpw

        pltpu.sync_copy(idx_u_hbm.at[pl.ds(wid * n_chunks, n_chunks)], idx_u_v)
        pltpu.sync_copy(idx_p_hbm.at[pl.ds(wid * n_chunks, n_chunks)], idx_p_v)
        pltpu.sync_copy(w_hbm, w_v)
        pltpu.sync_copy(b_hbm, b_v)

        for c in range(n_chunks):
            for v in range(CHUNK // K):
                sl = pl.ds(v * K, K)
                sup_u_v[c, sl] = jnp.right_shift(idx_u_v[c, sl], 3)
                sup_p_v[c, sl] = jnp.right_shift(idx_p_v[c, sl], 3)

        bufs_u = (bu0, bu1)
        bufs_p = (bp0, bp1)
        sems = (sem0, sem1)

        def fire(c):
            du = pltpu.async_copy(ut_hbm.at[sup_u_v.at[c]],
                                  bufs_u[c % 2], sems[c % 2])
            dp = pltpu.async_copy(pt_hbm.at[sup_p_v.at[c]],
                                  bufs_p[c % 2], sems[c % 2])
            return du, dp

        wv = w_v[...]
        bv = b_v[...]
        lanes = lax.iota(jnp.int32, K)
        seven = jnp.full((K,), 7, jnp.int32)

        descs = [None] * n_chunks
        descs[0] = fire(0)
        for c in range(n_chunks):
            if c + 1 < n_chunks:
                descs[c + 1] = fire(c + 1)
            du, dp = descs[c]
            du.wait()
            dp.wait()
            bu = bufs_u[c % 2]
            bp = bufs_p[c % 2]
            for g in range(CHUNK // K):
                sl = pl.ds(g * K, K)
                rows = g * K + lanes
                off_u = jnp.left_shift(idx_u_v[c, sl] & seven, 4)
                off_p = jnp.left_shift(idx_p_v[c, sl] & seven, 4)
                acc = bv
                for k in range(K):
                    gu = plsc.load_gather(bu, [rows, off_u + k])
                    gp = plsc.load_gather(bp, [rows, off_p + k])
                    acc = acc + gu * gp * wv[k]
                out_v[pl.ds(c * CHUNK + g * K, K)] = acc

        pltpu.sync_copy(out_v, out_hbm.at[pl.ds(base, bpw)])

    return mf_kernel(idx_u2, idx_p2, user_sup, prob_sup, w16, b16)


def kernel(input_user, input_prob, user_table, prob_table, dense_w, dense_b):
    info = plsc.get_sparse_core_info()
    out = _mf_sc(
        input_user.reshape(-1),
        input_prob.reshape(-1),
        user_table.reshape(-1, 128),
        prob_table.reshape(-1, 128),
        dense_w.reshape(K),
        jnp.broadcast_to(dense_b, (K,)),
        info.num_cores,
        info.num_subcores,
    )
    return out.reshape(-1, 1)


# speedup vs baseline: 2.7013x; 1.0118x over previous
import functools

import jax
import jax.numpy as jnp
from jax import lax
from jax.experimental import pallas as pl
from jax.experimental.pallas import tpu as pltpu
from jax.experimental.pallas import tpu_sc as plsc

K = 16
CHUNK = 128


@functools.partial(jax.jit, static_argnames=("num_cores", "num_subcores"))
def _mf_sc(user_idx, prob_idx, user_table, prob_table, w16, b16,
           num_cores, num_subcores):
    batch = user_idx.shape[0]
    nw = num_cores * num_subcores
    bpw = batch // nw
    n_chunks = bpw // CHUNK

    idx_u2 = user_idx.reshape(nw * n_chunks, CHUNK)
    idx_p2 = prob_idx.reshape(nw * n_chunks, CHUNK)

    mesh = plsc.VectorSubcoreMesh(
        core_axis_name="c", subcore_axis_name="s",
        num_cores=num_cores, num_subcores=num_subcores)

    @functools.partial(
        pl.kernel,
        out_type=jax.ShapeDtypeStruct((batch,), jnp.float32),
        mesh=mesh,
        scratch_types=[
            pltpu.VMEM((n_chunks, CHUNK), jnp.int32),
            pltpu.VMEM((n_chunks, CHUNK), jnp.int32),
            pltpu.VMEM((bpw, K), jnp.float32),
            pltpu.VMEM((bpw, K), jnp.float32),
            pltpu.VMEM((K,), jnp.float32),
            pltpu.VMEM((K,), jnp.float32),
            pltpu.VMEM((bpw,), jnp.float32),
            pltpu.SemaphoreType.DMA,
        ],
        compiler_params=pltpu.CompilerParams(
            needs_layout_passes=False, use_tc_tiling_on_sc=False),
    )
    def mf_kernel(idx_u_hbm, idx_p_hbm, ut_hbm, pt_hbm, w_hbm, b_hbm,
                  out_hbm, idx_u_v, idx_p_v, rows_u, rows_p, w_v, b_v,
                  out_v, sem):
        wid = lax.axis_index("s") * num_cores + lax.axis_index("c")
        base = wid * bpw

        pltpu.sync_copy(idx_u_hbm.at[pl.ds(wid * n_chunks, n_chunks)], idx_u_v)
        pltpu.sync_copy(idx_p_hbm.at[pl.ds(wid * n_chunks, n_chunks)], idx_p_v)
        pltpu.sync_copy(w_hbm, w_v)
        pltpu.sync_copy(b_hbm, b_v)

        copies = []
        for j in range(n_chunks):
            copies.append(pltpu.async_copy(
                ut_hbm.at[idx_u_v.at[j]],
                rows_u.at[pl.ds(j * CHUNK, CHUNK)], sem))
            copies.append(pltpu.async_copy(
                pt_hbm.at[idx_p_v.at[j]],
                rows_p.at[pl.ds(j * CHUNK, CHUNK)], sem))
        for c in copies:
            c.wait()

        wv = w_v[...]
        bv = b_v[...]
        lanes = lax.iota(jnp.int32, K)

        def body(g, carry):
            row_ids = g * K + lanes
            acc = bv
            for k in range(K):
                col = jnp.full((K,), k, jnp.int32)
                gu = plsc.load_gather(rows_u, [row_ids, col])
                gp = plsc.load_gather(rows_p, [row_ids, col])
                acc = acc + gu * gp * wv[k]
            out_v[pl.ds(g * K, K)] = acc
            return carry

        lax.fori_loop(0, bpw // K, body, 0)

        pltpu.sync_copy(out_v, out_hbm.at[pl.ds(base, bpw)])

    return mf_kernel(idx_u2, idx_p2, user_table, prob_table, w16, b16)


def kernel(input_user, input_prob, user_table, prob_table, dense_w, dense_b):
    info = plsc.get_sparse_core_info()
    out = _mf_sc(
        input_user.reshape(-1),
        input_prob.reshape(-1),
        user_table,
        prob_table,
        dense_w.reshape(K),
        jnp.broadcast_to(dense_b, (K,)),
        info.num_cores,
        info.num_subcores,
    )
    return out.reshape(-1, 1)
